# baseline (speedup 1.0000x reference)
import functools

import jax
import jax.numpy as jnp
from jax import lax
from jax.experimental import pallas as pl
from jax.experimental.pallas import tpu as pltpu, tpu_sc as plsc

_NC, _NS, _L = 2, 16, 16
_NW = _NC * _NS
_B = 4096
_DH = 2048
_RPW = _B // _NW
_NG = _RPW // _L
_NB = 512
_NPASS = 3
_BIG = 3.0e38

_mesh = plsc.VectorSubcoreMesh(core_axis_name="c", subcore_axis_name="s")


@functools.partial(
    pl.kernel,
    mesh=_mesh,
    out_type=jax.ShapeDtypeStruct((_B,), jnp.float32),
    compiler_params=pltpu.CompilerParams(needs_layout_passes=False),
    scratch_types=[
        pltpu.VMEM((_L * _DH,), jnp.float32),
        pltpu.VMEM((_DH,), jnp.float32),
        pltpu.VMEM((2 * _DH,), jnp.float32),
        pltpu.VMEM((_L,), jnp.float32),
        pltpu.VMEM((_NB * _L,), jnp.float32),
        pltpu.VMEM((_RPW,), jnp.float32),
    ],
)
def _wos_sc(x_hbm, mask_hbm, w_hbm, bias_hbm, out_hbm,
            xv, mask_v, w_v, bias_v, hist, yv):
    wid = lax.axis_index("s") * _NC + lax.axis_index("c")
    pltpu.sync_copy(mask_hbm, mask_v)
    pltpu.sync_copy(w_hbm, w_v)
    pltpu.sync_copy(bias_hbm, bias_v)
    bias = bias_v[...]
    lane = lax.iota(jnp.int32, _L)
    gidx0 = lane * _DH
    zeros = jnp.zeros((_L,), jnp.float32)

    _CB = _DH // _L

    for g in range(_NG):
        base = (wid * _RPW + g * _L) * _DH
        pltpu.sync_copy(x_hbm.at[pl.ds(base, _L * _DH)], xv)

        def cols(j):
            cbase = jnp.broadcast_to(j * _L, (_L,))
            out = []
            for k in range(_L):
                idxk = cbase + k
                v = (plsc.load_gather(xv, [gidx0 + idxk])
                     + plsc.load_gather(mask_v, [idxk]))
                out.append((idxk, v))
            return out

        def p_max(j, amax):
            for _, v in cols(j):
                amax = jnp.maximum(amax, jnp.abs(v))
            return amax
        amax = lax.fori_loop(0, _CB, p_max, zeros)
        lo = -amax - 1e-6
        hi = amax + 1e-6

        scale = None
        bhat = None
        for p in range(_NPASS):
            def p_zero(b, _):
                hist[pl.ds(b * _L, _L)] = zeros
                return 0
            lax.fori_loop(0, _NB, p_zero, 0)

            scale = _NB / (hi - lo)

            def p_hist(j, _, lo=lo, scale=scale):
                for idxk, v in cols(j):
                    wp = plsc.load_gather(w_v, [idxk])
                    wn = plsc.load_gather(w_v, [idxk + _DH])
                    b1 = jnp.clip((v - lo) * scale, 0.0, _NB - 1.0).astype(jnp.int32)
                    plsc.addupdate_scatter(hist, [b1 * _L + lane], wp)
                    b2 = jnp.clip((-v - lo) * scale, 0.0, _NB - 1.0).astype(jnp.int32)
                    plsc.addupdate_scatter(hist, [b2 * _L + lane], wn)
                return 0
            lax.fori_loop(0, _CB, p_hist, 0)

            def p_scan(i, carry):
                acc, bhat = carry
                b = _NB - 2 - i
                acc = acc + hist[pl.ds((b + 1) * _L, _L)]
                bhat = jnp.where(acc <= bias, b, bhat)
                return acc, bhat
            _, bhat = lax.fori_loop(
                0, _NB - 1, p_scan,
                (zeros, jnp.full((_L,), _NB - 1, jnp.int32)))
            if p < _NPASS - 1:
                bw = (hi - lo) * (1.0 / _NB)
                lo = lo + bhat.astype(jnp.float32) * bw
                hi = lo + bw

        def p_min(j, ymin, lo=lo, scale=scale, bhat=bhat):
            for _, v in cols(j):
                b1 = jnp.clip((v - lo) * scale, 0.0, float(_NB)).astype(jnp.int32)
                b2 = jnp.clip((-v - lo) * scale, 0.0, float(_NB)).astype(jnp.int32)
                y1 = jnp.where(b1 > bhat, v, _BIG)
                y2 = jnp.where(b2 > bhat, -v, _BIG)
                ymin = jnp.minimum(ymin, jnp.minimum(y1, y2))
            return ymin
        ymin = lax.fori_loop(0, _CB, p_min, jnp.full((_L,), _BIG, jnp.float32))
        yv[pl.ds(g * _L, _L)] = ymin

    pltpu.sync_copy(yv, out_hbm.at[pl.ds(wid * _RPW, _RPW)])


def kernel(x, mask, weight, bias):
    B = x.shape[0]
    xf = x.reshape(B * _DH).astype(jnp.float32)
    y = _wos_sc(xf, mask.reshape(-1), weight.reshape(-1),
                jnp.full((_L,), bias[0, 0], jnp.float32))
    return y.reshape(B, 1, 1, 1)


# speedup vs baseline: 1.1277x; 1.1277x over previous
import functools

import jax
import jax.numpy as jnp
from jax import lax
from jax.experimental import pallas as pl
from jax.experimental.pallas import tpu as pltpu, tpu_sc as plsc

_NC, _NS, _L = 2, 16, 16
_NW = _NC * _NS
_B = 4096
_DH = 2048
_RPW = _B // _NW
_NG = _RPW // _L
_NB = 512
_NPASS = 3
_BIG = 3.0e38

_mesh = plsc.VectorSubcoreMesh(core_axis_name="c", subcore_axis_name="s")


@functools.partial(
    pl.kernel,
    mesh=_mesh,
    out_type=jax.ShapeDtypeStruct((_B,), jnp.float32),
    compiler_params=pltpu.CompilerParams(needs_layout_passes=False),
    scratch_types=[
        pltpu.VMEM((_L * _DH,), jnp.float32),
        pltpu.VMEM((_DH,), jnp.float32),
        pltpu.VMEM((2 * _DH,), jnp.float32),
        pltpu.VMEM((_L,), jnp.float32),
        pltpu.VMEM((_NB * _L,), jnp.float32),
        pltpu.VMEM((_RPW,), jnp.float32),
    ],
)
def _wos_sc(x_hbm, mask_hbm, w_hbm, bias_hbm, out_hbm,
            xv, mask_v, w_v, bias_v, hist, yv):
    wid = lax.axis_index("s") * _NC + lax.axis_index("c")
    pltpu.sync_copy(mask_hbm, mask_v)
    pltpu.sync_copy(w_hbm, w_v)
    pltpu.sync_copy(bias_hbm, bias_v)
    bias = bias_v[...]
    lane = lax.iota(jnp.int32, _L)
    gidx0 = lane * _DH
    zeros = jnp.zeros((_L,), jnp.float32)

    _CB = _DH // _L

    for g in range(_NG):
        base = (wid * _RPW + g * _L) * _DH
        pltpu.sync_copy(x_hbm.at[pl.ds(base, _L * _DH)], xv)

        def cols(j):
            mv = mask_v[pl.ds(j * _L, _L)]
            return [plsc.load_gather(xv, [gidx0 + (j * _L + k)]) + mv[k]
                    for k in range(_L)]

        def p_max(j, amax):
            for v in cols(j):
                amax = jnp.maximum(amax, jnp.abs(v))
            return amax
        amax = lax.fori_loop(0, _CB, p_max, zeros)
        lo = -amax - 1e-6
        hi = amax + 1e-6

        scale = None
        bhat = None
        for p in range(_NPASS):
            def p_zero(b, _):
                hist[pl.ds(b * _L, _L)] = zeros
                return 0
            lax.fori_loop(0, _NB, p_zero, 0)

            scale = _NB / (hi - lo)

            def p_hist(j, _, lo=lo, scale=scale):
                wp = w_v[pl.ds(j * _L, _L)]
                wn = w_v[pl.ds(_DH + j * _L, _L)]
                for k, v in enumerate(cols(j)):
                    b1 = jnp.clip((v - lo) * scale, 0.0, _NB - 1.0).astype(jnp.int32)
                    plsc.addupdate_scatter(
                        hist, [b1 * _L + lane], jnp.broadcast_to(wp[k], (_L,)))
                    b2 = jnp.clip((-v - lo) * scale, 0.0, _NB - 1.0).astype(jnp.int32)
                    plsc.addupdate_scatter(
                        hist, [b2 * _L + lane], jnp.broadcast_to(wn[k], (_L,)))
                return 0
            lax.fori_loop(0, _CB, p_hist, 0)

            def p_scan(i, carry):
                acc, bhat = carry
                b = _NB - 2 - i
                acc = acc + hist[pl.ds((b + 1) * _L, _L)]
                bhat = jnp.where(acc <= bias, b, bhat)
                return acc, bhat
            _, bhat = lax.fori_loop(
                0, _NB - 1, p_scan,
                (zeros, jnp.full((_L,), _NB - 1, jnp.int32)))
            if p < _NPASS - 1:
                bw = (hi - lo) * (1.0 / _NB)
                lo = lo + bhat.astype(jnp.float32) * bw
                hi = lo + bw

        def p_min(j, ymin, lo=lo, scale=scale, bhat=bhat):
            for v in cols(j):
                b1 = jnp.clip((v - lo) * scale, 0.0, float(_NB)).astype(jnp.int32)
                b2 = jnp.clip((-v - lo) * scale, 0.0, float(_NB)).astype(jnp.int32)
                y1 = jnp.where(b1 > bhat, v, _BIG)
                y2 = jnp.where(b2 > bhat, -v, _BIG)
                ymin = jnp.minimum(ymin, jnp.minimum(y1, y2))
            return ymin
        ymin = lax.fori_loop(0, _CB, p_min, jnp.full((_L,), _BIG, jnp.float32))
        yv[pl.ds(g * _L, _L)] = ymin

    pltpu.sync_copy(yv, out_hbm.at[pl.ds(wid * _RPW, _RPW)])


def kernel(x, mask, weight, bias):
    B = x.shape[0]
    xf = x.reshape(B * _DH).astype(jnp.float32)
    y = _wos_sc(xf, mask.reshape(-1), weight.reshape(-1),
                jnp.full((_L,), bias[0, 0], jnp.float32))
    return y.reshape(B, 1, 1, 1)


# speedup vs baseline: 1.1867x; 1.0524x over previous
import functools

import jax
import jax.numpy as jnp
from jax import lax
from jax.experimental import pallas as pl
from jax.experimental.pallas import tpu as pltpu, tpu_sc as plsc

_NC, _NS, _L = 2, 16, 16
_NW = _NC * _NS
_B = 4096
_DH = 2048
_RPW = _B // _NW
_NG = _RPW // _L
_NB = 512
_NPASS = 3
_BIG = 3.0e38

_mesh = plsc.VectorSubcoreMesh(core_axis_name="c", subcore_axis_name="s")


@functools.partial(
    pl.kernel,
    mesh=_mesh,
    out_type=jax.ShapeDtypeStruct((_B,), jnp.float32),
    compiler_params=pltpu.CompilerParams(needs_layout_passes=False),
    scratch_types=[
        pltpu.VMEM((_L * _DH,), jnp.float32),
        pltpu.VMEM((_DH,), jnp.float32),
        pltpu.VMEM((2 * _DH,), jnp.float32),
        pltpu.VMEM((_L,), jnp.float32),
        pltpu.VMEM((_NB * _L,), jnp.float32),
        pltpu.VMEM((_RPW,), jnp.float32),
    ],
)
def _wos_sc(x_hbm, mask_hbm, w_hbm, bias_hbm, out_hbm,
            xv, mask_v, w_v, bias_v, hist, yv):
    wid = lax.axis_index("s") * _NC + lax.axis_index("c")
    pltpu.sync_copy(mask_hbm, mask_v)
    pltpu.sync_copy(w_hbm, w_v)
    pltpu.sync_copy(bias_hbm, bias_v)
    bias = bias_v[...]
    lane = lax.iota(jnp.int32, _L)
    gidx0 = lane * _DH
    zeros = jnp.zeros((_L,), jnp.float32)

    _CB = _DH // _L

    for g in range(_NG):
        base = (wid * _RPW + g * _L) * _DH
        pltpu.sync_copy(x_hbm.at[pl.ds(base, _L * _DH)], xv)

        def cols(j):
            mv = mask_v[pl.ds(j * _L, _L)]
            return [plsc.load_gather(xv, [gidx0 + (j * _L + k)]) + mv[k]
                    for k in range(_L)]

        @plsc.parallel_loop(0, _CB, carry=zeros)
        def amax(j, acc):
            for v in cols(j):
                acc = jnp.maximum(acc, jnp.abs(v))
            return acc
        lo = -amax - 1e-6
        hi = amax + 1e-6

        scale = None
        bhat = None
        for p in range(_NPASS):
            @plsc.parallel_loop(0, _NB // _L)
            def _(b):
                for kk in range(_L):
                    hist[pl.ds((b * _L + kk) * _L, _L)] = zeros

            scale = _NB / (hi - lo)

            def p_hist(j, lo=lo, scale=scale):
                wp = w_v[pl.ds(j * _L, _L)]
                wn = w_v[pl.ds(_DH + j * _L, _L)]
                for k, v in enumerate(cols(j)):
                    b1 = jnp.clip((v - lo) * scale, 0.0, _NB - 1.0).astype(jnp.int32)
                    plsc.addupdate_scatter(
                        hist, [b1 * _L + lane], jnp.broadcast_to(wp[k], (_L,)))
                    b2 = jnp.clip((-v - lo) * scale, 0.0, _NB - 1.0).astype(jnp.int32)
                    plsc.addupdate_scatter(
                        hist, [b2 * _L + lane], jnp.broadcast_to(wn[k], (_L,)))
            plsc.parallel_loop(0, _CB)(p_hist)

            def p_scan(i, carry):
                acc, bhat = carry
                b = _NB - 2 - i
                acc = acc + hist[pl.ds((b + 1) * _L, _L)]
                bhat = jnp.where(acc <= bias, b, bhat)
                return acc, bhat
            _, bhat = lax.fori_loop(
                0, _NB - 1, p_scan,
                (zeros, jnp.full((_L,), _NB - 1, jnp.int32)), unroll=8)
            if p < _NPASS - 1:
                bw = (hi - lo) * (1.0 / _NB)
                lo = lo + bhat.astype(jnp.float32) * bw
                hi = lo + bw

        @plsc.parallel_loop(0, _CB, carry=jnp.full((_L,), _BIG, jnp.float32))
        def ymin(j, acc, lo=lo, scale=scale, bhat=bhat):
            for v in cols(j):
                b1 = jnp.clip((v - lo) * scale, 0.0, float(_NB)).astype(jnp.int32)
                b2 = jnp.clip((-v - lo) * scale, 0.0, float(_NB)).astype(jnp.int32)
                y1 = jnp.where(b1 > bhat, v, _BIG)
                y2 = jnp.where(b2 > bhat, -v, _BIG)
                acc = jnp.minimum(acc, jnp.minimum(y1, y2))
            return acc
        yv[pl.ds(g * _L, _L)] = ymin

    pltpu.sync_copy(yv, out_hbm.at[pl.ds(wid * _RPW, _RPW)])


def kernel(x, mask, weight, bias):
    B = x.shape[0]
    xf = x.reshape(B * _DH).astype(jnp.float32)
    y = _wos_sc(xf, mask.reshape(-1), weight.reshape(-1),
                jnp.full((_L,), bias[0, 0], jnp.float32))
    return y.reshape(B, 1, 1, 1)


# speedup vs baseline: 1.6196x; 1.3648x over previous
import functools

import jax
import jax.numpy as jnp
from jax import lax
from jax.experimental import pallas as pl
from jax.experimental.pallas import tpu as pltpu, tpu_sc as plsc

_NC, _NS, _L = 2, 16, 16
_NW = _NC * _NS
_B = 4096
_DH = 2048
_RPW = _B // _NW
_NG = _RPW // _L
_NB = 512
_CB = _DH // _L
_BIG = 3.0e38

_mesh = plsc.VectorSubcoreMesh(core_axis_name="c", subcore_axis_name="s")


@functools.partial(
    pl.kernel,
    mesh=_mesh,
    out_type=jax.ShapeDtypeStruct((_B,), jnp.float32),
    compiler_params=pltpu.CompilerParams(needs_layout_passes=False),
    scratch_types=[
        pltpu.VMEM((_L * _DH,), jnp.float32),
        pltpu.VMEM((_DH,), jnp.float32),
        pltpu.VMEM((2 * _DH,), jnp.float32),
        pltpu.VMEM((_L,), jnp.float32),
        pltpu.VMEM((_NB * _L,), jnp.float32),
        pltpu.VMEM((2 * _DH * _L,), jnp.int32),
        pltpu.VMEM((_RPW,), jnp.float32),
    ],
)
def _wos_sc(x_hbm, mask_hbm, w_hbm, bias_hbm, out_hbm,
            xv, mask_v, w_v, bias_v, hist, clist, yv):
    wid = lax.axis_index("s") * _NC + lax.axis_index("c")
    pltpu.sync_copy(mask_hbm, mask_v)
    pltpu.sync_copy(w_hbm, w_v)
    pltpu.sync_copy(bias_hbm, bias_v)
    bias = bias_v[...]
    lane = lax.iota(jnp.int32, _L)
    gidx0 = lane * _DH
    zeros = jnp.zeros((_L,), jnp.float32)
    izeros = jnp.zeros((_L,), jnp.int32)
    bigs = jnp.full((_L,), _BIG, jnp.float32)

    def code_of(v):
        b = lax.bitcast_convert_type(v, jnp.int32)
        return b ^ ((b >> 31) & jnp.int32(0x7FFFFFFF))

    def zero_hist():
        @plsc.parallel_loop(0, _NB // _L)
        def _(b):
            for kk in range(_L):
                hist[pl.ds((b * _L + kk) * _L, _L)] = zeros

    def scan_hist(thr):
        def p_scan(i, carry):
            acc, bh, sab = carry
            b = _NB - 2 - i
            acc = acc + hist[pl.ds((b + 1) * _L, _L)]
            cand = acc <= thr
            bh = jnp.where(cand, b, bh)
            sab = jnp.where(cand, acc, sab)
            return acc, bh, sab
        _, bh, sab = lax.fori_loop(
            0, _NB - 1, p_scan,
            (zeros, jnp.full((_L,), _NB - 1, jnp.int32), zeros), unroll=8)
        return bh, sab

    def group_body(g, _):
        base = (wid * _RPW + g * _L) * _DH
        pltpu.sync_copy(x_hbm.at[pl.ds(base, _L * _DH)], xv)

        zero_hist()

        def p_hist(j):
            mv = mask_v[pl.ds(j * _L, _L)]
            wp = w_v[pl.ds(j * _L, _L)]
            wn = w_v[pl.ds(_DH + j * _L, _L)]
            for k in range(_L):
                c = j * _L + k
                v = plsc.load_gather(xv, [gidx0 + c]) + mv[k]
                m = code_of(v)
                b1 = (m >> 23) + 256
                plsc.addupdate_scatter(
                    hist, [b1 * _L + lane], jnp.broadcast_to(wp[k], (_L,)))
                b2 = ((~m) >> 23) + 256
                plsc.addupdate_scatter(
                    hist, [b2 * _L + lane], jnp.broadcast_to(wn[k], (_L,)))
        plsc.parallel_loop(0, _CB)(p_hist)

        bhat1, sab = scan_hist(bias)
        bias2 = bias - sab
        win = (bhat1 - 256) << 23

        @plsc.parallel_loop(0, _CB, carry=(izeros, bigs))
        def cnt_mab(j, carry):
            cnt, mab = carry
            mv = mask_v[pl.ds(j * _L, _L)]
            for k in range(_L):
                c = j * _L + k
                v = plsc.load_gather(xv, [gidx0 + c]) + mv[k]
                m = code_of(v)
                cidx = jnp.broadcast_to(c, (_L,))
                b1 = (m >> 23) + 256
                plsc.store_scatter(clist, [lane + cnt * _L], cidx,
                                   mask=b1 == bhat1)
                cnt = cnt + (b1 == bhat1).astype(jnp.int32)
                mab = jnp.minimum(mab, jnp.where(b1 > bhat1, v, _BIG))
                b2 = ((~m) >> 23) + 256
                plsc.store_scatter(clist, [lane + cnt * _L], cidx + _DH,
                                   mask=b2 == bhat1)
                cnt = cnt + (b2 == bhat1).astype(jnp.int32)
                mab = jnp.minimum(mab, jnp.where(b2 > bhat1, -v, _BIG))
            return cnt, mab
        cnt, mabove = cnt_mab
        cmax = jnp.max(cnt)

        def listvals(i):
            ci = clist[pl.ds(i * _L, _L)] & (2 * _DH - 1)
            cc = ci & (_DH - 1)
            xmv = (plsc.load_gather(xv, [gidx0 + cc])
                   + plsc.load_gather(mask_v, [cc]))
            neg = ci >= _DH
            val = jnp.where(neg, -xmv, xmv)
            mplus = code_of(xmv)
            m = jnp.where(neg, ~mplus, mplus)
            return ci, val, m

        bhat = None
        sh = None
        for sh_ in (14, 5):
            sh = sh_
            zero_hist()
            wsh = win >> sh

            def p_rh(i, wsh=wsh, sh=sh):
                ci, _, m = listvals(i)
                wv = plsc.load_gather(w_v, [ci])
                valid = jnp.broadcast_to(i, (_L,)) < cnt
                b = jnp.clip((m >> sh) - wsh, 0, _NB - 1)
                plsc.addupdate_scatter(hist, [b * _L + lane], wv, mask=valid)
            plsc.parallel_loop(0, cmax)(p_rh)

            bhat, _ = scan_hist(bias2)
            if sh == 14:
                win = win + (bhat << sh)

        wsh = win >> sh

        @plsc.parallel_loop(0, cmax, carry=bigs)
        def m_in(i, acc, wsh=wsh, sh=sh, bhat=bhat):
            _, val, m = listvals(i)
            valid = jnp.broadcast_to(i, (_L,)) < cnt
            br = jnp.clip((m >> sh) - wsh, 0, _NB)
            ok = valid & (br > bhat)
            return jnp.minimum(acc, jnp.where(ok, val, _BIG))
        yv[pl.ds(g * _L, _L)] = jnp.minimum(m_in, mabove)
        return 0

    lax.fori_loop(0, _NG, group_body, 0)
    pltpu.sync_copy(yv, out_hbm.at[pl.ds(wid * _RPW, _RPW)])


def kernel(x, mask, weight, bias):
    B = x.shape[0]
    xf = x.reshape(B * _DH).astype(jnp.float32)
    y = _wos_sc(xf, mask.reshape(-1), weight.reshape(-1),
                jnp.full((_L,), bias[0, 0], jnp.float32))
    return y.reshape(B, 1, 1, 1)


# speedup vs baseline: 1.6288x; 1.0057x over previous
import functools

import jax
import jax.numpy as jnp
from jax import lax
from jax.experimental import pallas as pl
from jax.experimental.pallas import tpu as pltpu, tpu_sc as plsc

_NC, _NS, _L = 2, 16, 16
_NW = _NC * _NS
_B = 4096
_DH = 2048
_RPW = _B // _NW
_NG = _RPW // _L
_NB = 512
_CB = _DH // _L
_BIG = 3.0e38

_mesh = plsc.VectorSubcoreMesh(core_axis_name="c", subcore_axis_name="s")


@functools.partial(
    pl.kernel,
    mesh=_mesh,
    out_type=jax.ShapeDtypeStruct((_B,), jnp.float32),
    compiler_params=pltpu.CompilerParams(needs_layout_passes=False),
    scratch_types=[
        pltpu.VMEM((_L * _DH,), jnp.float32),
        pltpu.VMEM((_DH,), jnp.float32),
        pltpu.VMEM((2 * _DH,), jnp.float32),
        pltpu.VMEM((_L,), jnp.float32),
        pltpu.VMEM((_NB * _L,), jnp.float32),
        pltpu.VMEM((_NB * _L,), jnp.float32),
        pltpu.VMEM((_DH * _L,), jnp.int32),
        pltpu.VMEM((_DH * _L,), jnp.int32),
        pltpu.VMEM((_RPW,), jnp.float32),
    ],
)
def _wos_sc(x_hbm, mask_hbm, w_hbm, bias_hbm, out_hbm,
            xv, mask_v, w_v, bias_v, hista, histb, clp, cln, yv):
    wid = lax.axis_index("s") * _NC + lax.axis_index("c")
    pltpu.sync_copy(mask_hbm, mask_v)
    pltpu.sync_copy(w_hbm, w_v)
    pltpu.sync_copy(bias_hbm, bias_v)
    bias = bias_v[...]
    lane = lax.iota(jnp.int32, _L)
    gidx0 = lane * _DH
    idx1v = lane + 256 * _L
    idx2v = lane + 255 * _L
    zeros = jnp.zeros((_L,), jnp.float32)
    izeros = jnp.zeros((_L,), jnp.int32)
    bigs = jnp.full((_L,), _BIG, jnp.float32)

    def code_of(v):
        b = lax.bitcast_convert_type(v, jnp.int32)
        return b ^ ((b >> 31) & jnp.int32(0x7FFFFFFF))

    def zero_hist():
        @plsc.parallel_loop(0, _NB // _L)
        def _(b):
            for kk in range(_L):
                hista[pl.ds((b * _L + kk) * _L, _L)] = zeros
                histb[pl.ds((b * _L + kk) * _L, _L)] = zeros

    def scan_hist(thr):
        def p_scan(i, carry):
            acc, bh, sab = carry
            b = _NB - 2 - i
            acc = (acc + hista[pl.ds((b + 1) * _L, _L)]
                   + histb[pl.ds((b + 1) * _L, _L)])
            cand = acc <= thr
            bh = jnp.where(cand, b, bh)
            sab = jnp.where(cand, acc, sab)
            return acc, bh, sab
        _, bh, sab = lax.fori_loop(
            0, _NB - 1, p_scan,
            (zeros, jnp.full((_L,), _NB - 1, jnp.int32), zeros), unroll=8)
        return bh, sab

    def group_body(g, _):
        base = (wid * _RPW + g * _L) * _DH
        pltpu.sync_copy(x_hbm.at[pl.ds(base, _L * _DH)], xv)

        zero_hist()

        def p_hist(j):
            mv = mask_v[pl.ds(j * _L, _L)]
            wp = w_v[pl.ds(j * _L, _L)]
            wn = w_v[pl.ds(_DH + j * _L, _L)]
            for k in range(_L):
                c = j * _L + k
                v = plsc.load_gather(xv, [gidx0 + c]) + mv[k]
                e16 = (code_of(v) >> 23) << 4
                plsc.addupdate_scatter(
                    hista, [idx1v + e16], jnp.broadcast_to(wp[k], (_L,)))
                plsc.addupdate_scatter(
                    histb, [idx2v - e16], jnp.broadcast_to(wn[k], (_L,)))
        plsc.parallel_loop(0, _CB)(p_hist)

        bhat1, sab = scan_hist(bias)
        bias2 = bias - sab
        win = (bhat1 - 256) << 23
        ebp = bhat1 - 256
        ebn = 255 - bhat1

        @plsc.parallel_loop(0, _CB, carry=(izeros, izeros, bigs))
        def cnts(j, carry):
            cp, cn, mab = carry
            mv = mask_v[pl.ds(j * _L, _L)]
            for k in range(_L):
                c = j * _L + k
                v = plsc.load_gather(xv, [gidx0 + c]) + mv[k]
                e = code_of(v) >> 23
                cidx = jnp.broadcast_to(c, (_L,))
                plsc.store_scatter(clp, [lane + cp * _L], cidx, mask=e == ebp)
                cp = cp + (e == ebp).astype(jnp.int32)
                mab = jnp.minimum(mab, jnp.where(e > ebp, v, _BIG))
                plsc.store_scatter(cln, [lane + cn * _L], cidx, mask=e == ebn)
                cn = cn + (e == ebn).astype(jnp.int32)
                mab = jnp.minimum(mab, jnp.where(e < ebn, -v, _BIG))
            return cp, cn, mab
        cntp, cntn, mabove = cnts
        cmaxp = jnp.max(cntp)
        cmaxn = jnp.max(cntn)

        def listval(clref, i):
            cc = clref[pl.ds(i * _L, _L)] & (_DH - 1)
            xmv = (plsc.load_gather(xv, [gidx0 + cc])
                   + plsc.load_gather(mask_v, [cc]))
            return cc, xmv

        bhat = None
        sh = None
        for sh_ in (14, 5):
            sh = sh_
            zero_hist()
            wsh = win >> sh

            def p_rhp(i, wsh=wsh, sh=sh):
                cc, xmv = listval(clp, i)
                wv = plsc.load_gather(w_v, [cc])
                valid = jnp.broadcast_to(i, (_L,)) < cntp
                b = jnp.clip((code_of(xmv) >> sh) - wsh, 0, _NB - 1)
                plsc.addupdate_scatter(hista, [b * _L + lane], wv, mask=valid)
            plsc.parallel_loop(0, cmaxp)(p_rhp)

            def p_rhn(i, wsh=wsh, sh=sh):
                cc, xmv = listval(cln, i)
                wv = plsc.load_gather(w_v, [cc + _DH])
                valid = jnp.broadcast_to(i, (_L,)) < cntn
                b = jnp.clip(((~code_of(xmv)) >> sh) - wsh, 0, _NB - 1)
                plsc.addupdate_scatter(histb, [b * _L + lane], wv, mask=valid)
            plsc.parallel_loop(0, cmaxn)(p_rhn)

            bhat, _ = scan_hist(bias2)
            if sh == 14:
                win = win + (bhat << sh)

        wsh = win >> sh

        @plsc.parallel_loop(0, cmaxp, carry=bigs)
        def m_inp(i, acc, wsh=wsh, sh=sh, bhat=bhat):
            cc, xmv = listval(clp, i)
            valid = jnp.broadcast_to(i, (_L,)) < cntp
            br = jnp.clip((code_of(xmv) >> sh) - wsh, 0, _NB)
            ok = valid & (br > bhat)
            return jnp.minimum(acc, jnp.where(ok, xmv, _BIG))

        @plsc.parallel_loop(0, cmaxn, carry=m_inp)
        def m_in(i, acc, wsh=wsh, sh=sh, bhat=bhat):
            cc, xmv = listval(cln, i)
            valid = jnp.broadcast_to(i, (_L,)) < cntn
            br = jnp.clip(((~code_of(xmv)) >> sh) - wsh, 0, _NB)
            ok = valid & (br > bhat)
            return jnp.minimum(acc, jnp.where(ok, -xmv, _BIG))

        yv[pl.ds(g * _L, _L)] = jnp.minimum(m_in, mabove)
        return 0

    lax.fori_loop(0, _NG, group_body, 0)
    pltpu.sync_copy(yv, out_hbm.at[pl.ds(wid * _RPW, _RPW)])


def kernel(x, mask, weight, bias):
    B = x.shape[0]
    xf = x.reshape(B * _DH).astype(jnp.float32)
    y = _wos_sc(xf, mask.reshape(-1), weight.reshape(-1),
                jnp.full((_L,), bias[0, 0], jnp.float32))
    return y.reshape(B, 1, 1, 1)


# speedup vs baseline: 4.4226x; 2.7153x over previous
import functools

import jax
import jax.numpy as jnp
from jax import lax
from jax.experimental import pallas as pl
from jax.experimental.pallas import tpu as pltpu, tpu_sc as plsc

_NC, _NS, _L = 2, 16, 16
_NW = _NC * _NS
_B = 4096
_DH = 2048
_RPW = _B // _NW
_NG = _RPW // _L
_NB = 512
_CB = _DH // _L
_BIG = 3.0e38

_mesh = plsc.VectorSubcoreMesh(core_axis_name="c", subcore_axis_name="s")


@functools.partial(
    pl.kernel,
    mesh=_mesh,
    out_type=jax.ShapeDtypeStruct((_B,), jnp.float32),
    compiler_params=pltpu.CompilerParams(needs_layout_passes=False, use_tc_tiling_on_sc=False),
    scratch_types=[
        pltpu.VMEM((_DH, _L), jnp.float32),
        pltpu.VMEM((_DH,), jnp.float32),
        pltpu.VMEM((2 * _DH,), jnp.float32),
        pltpu.VMEM((_L,), jnp.float32),
        pltpu.VMEM((_NB * _L,), jnp.float32),
        pltpu.VMEM((_NB * _L,), jnp.float32),
        pltpu.VMEM((_DH * _L,), jnp.int32),
        pltpu.VMEM((_DH * _L,), jnp.int32),
        pltpu.VMEM((_RPW,), jnp.float32),
    ],
)
def _wos_sc(x_hbm, mask_hbm, w_hbm, bias_hbm, out_hbm,
            xv, mask_v, w_v, bias_v, hista, histb, clp, cln, yv):
    wid = lax.axis_index("s") * _NC + lax.axis_index("c")
    pltpu.sync_copy(mask_hbm, mask_v)
    pltpu.sync_copy(w_hbm, w_v)
    pltpu.sync_copy(bias_hbm, bias_v)
    bias = bias_v[...]
    lane = lax.iota(jnp.int32, _L)
    gidx0 = lane * _DH
    idx1v = lane + 256 * _L
    idx2v = lane + 255 * _L
    zeros = jnp.zeros((_L,), jnp.float32)
    izeros = jnp.zeros((_L,), jnp.int32)
    bigs = jnp.full((_L,), _BIG, jnp.float32)

    def code_of(v):
        b = lax.bitcast_convert_type(v, jnp.int32)
        return b ^ ((b >> 31) & jnp.int32(0x7FFFFFFF))

    def zero_hist():
        @plsc.parallel_loop(0, _NB // _L)
        def _(b):
            for kk in range(_L):
                hista[pl.ds((b * _L + kk) * _L, _L)] = zeros
                histb[pl.ds((b * _L + kk) * _L, _L)] = zeros

    def scan_hist(thr):
        def p_scan(i, carry):
            acc, bh, sab = carry
            b = _NB - 2 - i
            acc = (acc + hista[pl.ds((b + 1) * _L, _L)]
                   + histb[pl.ds((b + 1) * _L, _L)])
            cand = acc <= thr
            bh = jnp.where(cand, b, bh)
            sab = jnp.where(cand, acc, sab)
            return acc, bh, sab
        _, bh, sab = lax.fori_loop(
            0, _NB - 1, p_scan,
            (zeros, jnp.full((_L,), _NB - 1, jnp.int32), zeros), unroll=8)
        return bh, sab

    def group_body(g, _):
        base = wid * _RPW + g * _L
        pltpu.sync_copy(x_hbm.at[:, pl.ds(base, _L)], xv)

        zero_hist()

        def p_hist(j):
            mv = mask_v[pl.ds(j * _L, _L)]
            wp = w_v[pl.ds(j * _L, _L)]
            wn = w_v[pl.ds(_DH + j * _L, _L)]
            for k in range(_L):
                c = j * _L + k
                v = xv[c] + mv[k]
                e16 = (code_of(v) >> 23) << 4
                plsc.addupdate_scatter(
                    hista, [idx1v + e16], jnp.broadcast_to(wp[k], (_L,)))
                plsc.addupdate_scatter(
                    histb, [idx2v - e16], jnp.broadcast_to(wn[k], (_L,)))
        plsc.parallel_loop(0, _CB)(p_hist)

        bhat1, sab = scan_hist(bias)
        bias2 = bias - sab
        win = (bhat1 - 256) << 23
        ebp = bhat1 - 256
        ebn = 255 - bhat1

        @plsc.parallel_loop(0, _CB, carry=(izeros, izeros, bigs))
        def cnts(j, carry):
            cp, cn, mab = carry
            mv = mask_v[pl.ds(j * _L, _L)]
            for k in range(_L):
                c = j * _L + k
                v = xv[c] + mv[k]
                e = code_of(v) >> 23
                cidx = jnp.broadcast_to(c, (_L,))
                plsc.store_scatter(clp, [lane + cp * _L], cidx, mask=e == ebp)
                cp = cp + (e == ebp).astype(jnp.int32)
                mab = jnp.minimum(mab, jnp.where(e > ebp, v, _BIG))
                plsc.store_scatter(cln, [lane + cn * _L], cidx, mask=e == ebn)
                cn = cn + (e == ebn).astype(jnp.int32)
                mab = jnp.minimum(mab, jnp.where(e < ebn, -v, _BIG))
            return cp, cn, mab
        cntp, cntn, mabove = cnts
        cmaxp = jnp.max(cntp)
        cmaxn = jnp.max(cntn)

        def listval(clref, i):
            cc = clref[pl.ds(i * _L, _L)] & (_DH - 1)
            xmv = (plsc.load_gather(xv, [cc, lane])
                   + plsc.load_gather(mask_v, [cc]))
            return cc, xmv

        bhat = None
        sh = None
        for sh_ in (14, 5):
            sh = sh_
            zero_hist()
            wsh = win >> sh

            def p_rhp(i, wsh=wsh, sh=sh):
                cc, xmv = listval(clp, i)
                wv = plsc.load_gather(w_v, [cc])
                valid = jnp.broadcast_to(i, (_L,)) < cntp
                b = jnp.clip((code_of(xmv) >> sh) - wsh, 0, _NB - 1)
                plsc.addupdate_scatter(hista, [b * _L + lane], wv, mask=valid)
            plsc.parallel_loop(0, cmaxp)(p_rhp)

            def p_rhn(i, wsh=wsh, sh=sh):
                cc, xmv = listval(cln, i)
                wv = plsc.load_gather(w_v, [cc + _DH])
                valid = jnp.broadcast_to(i, (_L,)) < cntn
                b = jnp.clip(((~code_of(xmv)) >> sh) - wsh, 0, _NB - 1)
                plsc.addupdate_scatter(histb, [b * _L + lane], wv, mask=valid)
            plsc.parallel_loop(0, cmaxn)(p_rhn)

            bhat, _ = scan_hist(bias2)
            if sh == 14:
                win = win + (bhat << sh)

        wsh = win >> sh

        @plsc.parallel_loop(0, cmaxp, carry=bigs)
        def m_inp(i, acc, wsh=wsh, sh=sh, bhat=bhat):
            cc, xmv = listval(clp, i)
            valid = jnp.broadcast_to(i, (_L,)) < cntp
            br = jnp.clip((code_of(xmv) >> sh) - wsh, 0, _NB)
            ok = valid & (br > bhat)
            return jnp.minimum(acc, jnp.where(ok, xmv, _BIG))

        @plsc.parallel_loop(0, cmaxn, carry=m_inp)
        def m_in(i, acc, wsh=wsh, sh=sh, bhat=bhat):
            cc, xmv = listval(cln, i)
            valid = jnp.broadcast_to(i, (_L,)) < cntn
            br = jnp.clip(((~code_of(xmv)) >> sh) - wsh, 0, _NB)
            ok = valid & (br > bhat)
            return jnp.minimum(acc, jnp.where(ok, -xmv, _BIG))

        yv[pl.ds(g * _L, _L)] = jnp.minimum(m_in, mabove)
        return 0

    lax.fori_loop(0, _NG, group_body, 0)
    pltpu.sync_copy(yv, out_hbm.at[pl.ds(wid * _RPW, _RPW)])


def kernel(x, mask, weight, bias):
    B = x.shape[0]
    xt = x.reshape(B, _DH).astype(jnp.float32).T
    y = _wos_sc(xt, mask.reshape(-1), weight.reshape(-1),
                jnp.full((_L,), bias[0, 0], jnp.float32))
    return y.reshape(B, 1, 1, 1)
